# baseline (speedup 1.0000x reference)
import jax
import jax.numpy as jnp
from jax import lax
from jax.experimental import pallas as pl
from jax.experimental.pallas import tpu as pltpu
from jax.experimental.pallas import tpu_sc as plsc

_B = 4
_S = 2048
_D = 1024
_NC = 2
_NS = 16
_NW = _NC * _NS
_SPW = _S // _NW
_CH = 8
_NQ = _SPW // _CH
_VPQ = _CH * _D // 16


def _body(x_hbm, tok_hbm, pos_hbm, out_hbm,
          idx_v, p0, p1,
          g00, g01, g02, g03, g10, g11, g12, g13,
          psem0, psem1, gsem0, gsem1, ssem0, ssem1):
    pbufs = (p0, p1)
    gbufs = ((g00, g01, g02, g03), (g10, g11, g12, g13))
    psems = (psem0, psem1)
    gsems = (gsem0, gsem1)
    ssems = (ssem0, ssem1)

    wid = lax.axis_index("s") * _NC + lax.axis_index("c")
    s0 = wid * _SPW

    for b in range(_B):
        pltpu.sync_copy(x_hbm.at[pl.ds(b * _S + s0, _SPW)], idx_v.at[b])

    def start_pos(q):
        return pltpu.async_copy(
            pos_hbm.at[pl.ds(s0 + q * _CH, _CH)], pbufs[q % 2], psems[q % 2])

    def start_quad_gathers(q):
        p = q % 2
        return [
            pltpu.async_copy(
                tok_hbm.at[idx_v.at[b, pl.ds(q * _CH, _CH)]],
                gbufs[p][b], gsems[p])
            for b in range(_B)
        ]

    ph = [start_pos(0), None]
    gh = [start_quad_gathers(0), start_quad_gathers(1)]
    sh = [None, None]

    for q in range(_NQ):
        p = q % 2
        if q + 1 < _NQ:
            ph[(q + 1) % 2] = start_pos(q + 1)
        ph[p].wait()
        for h in gh[p]:
            h.wait()
        pbuf = pbufs[p]
        bufs = gbufs[p]

        @plsc.parallel_loop(0, _VPQ, unroll=2)
        def _(i, bufs=bufs, pbuf=pbuf):
            r = i >> 6
            c = pl.multiple_of((i & 63) << 4, 16)
            pv = pbuf[r, pl.ds(c, 16)]
            for b in range(_B):
                bufs[b][r, pl.ds(c, 16)] += pv

        sh[p] = [
            pltpu.async_copy(
                bufs[b], out_hbm.at[pl.ds(b * _S + s0 + q * _CH, _CH)],
                ssems[p])
            for b in range(_B)
        ]
        if q + 2 < _NQ:
            for h in sh[p]:
                h.wait()
            gh[p] = start_quad_gathers(q + 2)

    for p in range(2):
        if sh[p] is not None:
            for h in sh[p]:
                h.wait()


def kernel(x, tok_table, pos_table):
    mesh = plsc.VectorSubcoreMesh(core_axis_name="c", subcore_axis_name="s")
    out = pl.kernel(
        _body,
        out_type=jax.ShapeDtypeStruct((_B * _S, _D), jnp.float32),
        mesh=mesh,
        scratch_types=[
            pltpu.VMEM((_B, _SPW), jnp.int32),
            pltpu.VMEM((_CH, _D), jnp.float32),
            pltpu.VMEM((_CH, _D), jnp.float32),
            pltpu.VMEM((_CH, _D), jnp.float32),
            pltpu.VMEM((_CH, _D), jnp.float32),
            pltpu.VMEM((_CH, _D), jnp.float32),
            pltpu.VMEM((_CH, _D), jnp.float32),
            pltpu.VMEM((_CH, _D), jnp.float32),
            pltpu.VMEM((_CH, _D), jnp.float32),
            pltpu.VMEM((_CH, _D), jnp.float32),
            pltpu.VMEM((_CH, _D), jnp.float32),
            pltpu.SemaphoreType.DMA,
            pltpu.SemaphoreType.DMA,
            pltpu.SemaphoreType.DMA,
            pltpu.SemaphoreType.DMA,
            pltpu.SemaphoreType.DMA,
            pltpu.SemaphoreType.DMA,
        ],
    )(x.reshape(-1).astype(jnp.int32), tok_table, pos_table)
    return out.reshape(_B, _S, _D)


# speedup vs baseline: 1.0471x; 1.0471x over previous
import jax
import jax.numpy as jnp
from jax import lax
from jax.experimental import pallas as pl
from jax.experimental.pallas import tpu as pltpu
from jax.experimental.pallas import tpu_sc as plsc

_B = 4
_S = 2048
_D = 1024
_NC = 2
_NS = 16
_NW = _NC * _NS
_SPW = _S // _NW
_CH = 8
_NQ = _SPW // _CH
_NR = 3
_VPQ = _CH * _D // 16


def _body(x_hbm, tok_hbm, pos_hbm, out_hbm,
          idx_v, p0, p1, p2,
          g00, g01, g02, g03, g10, g11, g12, g13, g20, g21, g22, g23,
          isem, psem0, psem1, psem2, gsem0, gsem1, gsem2,
          ssem0, ssem1, ssem2):
    pbufs = (p0, p1, p2)
    gbufs = ((g00, g01, g02, g03), (g10, g11, g12, g13), (g20, g21, g22, g23))
    psems = (psem0, psem1, psem2)
    gsems = (gsem0, gsem1, gsem2)
    ssems = (ssem0, ssem1, ssem2)

    wid = lax.axis_index("s") * _NC + lax.axis_index("c")
    s0 = wid * _SPW

    ih = [pltpu.async_copy(x_hbm.at[b, pl.ds(s0, _SPW)], idx_v.at[b], isem)
          for b in range(_B)]
    for h in ih:
        h.wait()

    def start_pos(q):
        k = q % _NR
        return pltpu.async_copy(
            pos_hbm.at[pl.ds(s0 + q * _CH, _CH)], pbufs[k], psems[k])

    def start_quad_gathers(q):
        k = q % _NR
        return [
            pltpu.async_copy(
                tok_hbm.at[idx_v.at[b, pl.ds(q * _CH, _CH)]],
                gbufs[k][b], gsems[k])
            for b in range(_B)
        ]

    ph = [start_pos(0), start_pos(1), start_pos(2)]
    gh = [start_quad_gathers(0), start_quad_gathers(1), start_quad_gathers(2)]
    sh = [None, None, None]

    for q in range(_NQ):
        k = q % _NR
        ph[k].wait()
        for h in gh[k]:
            h.wait()
        pbuf = pbufs[k]
        bufs = gbufs[k]

        @plsc.parallel_loop(0, _VPQ, unroll=2)
        def _(i, bufs=bufs, pbuf=pbuf):
            r = i >> 6
            c = pl.multiple_of((i & 63) << 4, 16)
            pv = pbuf[r, pl.ds(c, 16)]
            for b in range(_B):
                bufs[b][r, pl.ds(c, 16)] += pv

        sh[k] = [
            pltpu.async_copy(
                bufs[b], out_hbm.at[pl.ds(b * _S + s0 + q * _CH, _CH)],
                ssems[k])
            for b in range(_B)
        ]
        if q + _NR < _NQ:
            for h in sh[k]:
                h.wait()
            ph[k] = start_pos(q + _NR)
            gh[k] = start_quad_gathers(q + _NR)

    for k in range(_NR):
        if sh[k] is not None:
            for h in sh[k]:
                h.wait()


def kernel(x, tok_table, pos_table):
    mesh = plsc.VectorSubcoreMesh(core_axis_name="c", subcore_axis_name="s")
    vmem = [pltpu.VMEM((_CH, _D), jnp.float32)] * (_NR + _NR * _B)
    out = pl.kernel(
        _body,
        out_type=jax.ShapeDtypeStruct((_B * _S, _D), jnp.float32),
        mesh=mesh,
        scratch_types=(
            [pltpu.VMEM((_B, _SPW), jnp.int32)]
            + vmem
            + [pltpu.SemaphoreType.DMA] * 10
        ),
    )(x.astype(jnp.int32), tok_table, pos_table)
    return out.reshape(_B, _S, _D)
